# baseline (speedup 1.0000x reference)
import jax
import jax.numpy as jnp
from jax.experimental import pallas as pl
from jax.experimental.pallas import tpu as pltpu

_DEPTH = 384
_TILE = 96


def _body(x_ref, wt_ref, bias_ref, basis_ref, out_ref, map_ref, am_ref):
    c = pl.program_id(0)
    b = pl.program_id(1)
    num_caps = x_ref.shape[2]
    n_own = _DEPTH // _TILE

    x = x_ref[0, b]
    featT = jax.lax.dot_general(wt_ref[...], x, (((1,), (1,)), ((), ())),
                                preferred_element_type=jnp.float32)
    featT = featT + bias_ref[...]
    mT = featT * basis_ref[...]
    map_ref[0] = mT

    tmax = jnp.max(mT, axis=1, keepdims=True)
    e = jnp.exp(mT - tmax)
    den = jnp.sum(e, axis=1, keepdims=True)
    amax = 1.0 / den
    ef = e * featT
    caps_i = jax.lax.broadcasted_iota(jnp.int32, mT.shape, 1)

    @pl.when(c < n_own)
    def _():
        a = e / den
        tie = a == amax
        first = jnp.min(jnp.where(tie, caps_i, num_caps),
                        axis=1, keepdims=True)
        am_ref[b, pl.ds(c * _TILE, _TILE)] = first
        out_ref[0] = jnp.where(caps_i == first, ef, 0.0) * amax

    @pl.when(c >= n_own)
    def _():
        pair_start = jnp.clip(c - n_own, 0, n_own - 1) * _TILE
        stored = am_ref[b, pl.ds(pair_start, _TILE)]
        routed = jnp.where(caps_i == stored, ef, 0.0)
        out_ref[0] = jnp.where(c >= 2 * n_own, ef * (1.0 / num_caps),
                               routed) * amax


def kernel(feat_list, W, b, caps_basis):
    L, Bv, Nv = feat_list.shape[0], feat_list.shape[1], feat_list.shape[2]
    cin = feat_list.shape[-1]
    num_caps = caps_basis.shape[1]
    cout = caps_basis.shape[3]
    xs = feat_list.reshape(L, Bv, Nv * Nv, cin)
    slab = L - 2
    basisT = jnp.swapaxes(caps_basis.reshape(num_caps, cout), 0, 1)
    wT = jnp.swapaxes(W, 0, 1)
    biasT = b.reshape(cout, 1)
    n_c = cout // _TILE
    f32 = jnp.float32

    outT, mapT = pl.pallas_call(
        _body,
        grid=(n_c, Bv),
        in_specs=[
            pl.BlockSpec((1, Bv, num_caps, cin),
                         lambda c, bb: (slab, 0, 0, 0)),
            pl.BlockSpec((_TILE, cin), lambda c, bb: (c, 0)),
            pl.BlockSpec((_TILE, 1), lambda c, bb: (c, 0)),
            pl.BlockSpec((_TILE, num_caps), lambda c, bb: (c, 0)),
        ],
        out_specs=(
            pl.BlockSpec((1, _TILE, num_caps), lambda c, bb: (bb, c, 0)),
            pl.BlockSpec((1, _TILE, num_caps), lambda c, bb: (bb, c, 0)),
        ),
        out_shape=(
            jax.ShapeDtypeStruct((Bv, cout, num_caps), f32),
            jax.ShapeDtypeStruct((Bv, cout, num_caps), f32),
        ),
        scratch_shapes=[pltpu.VMEM((Bv, _DEPTH, 1), jnp.int32)],
        compiler_params=pltpu.CompilerParams(
            dimension_semantics=("arbitrary", "arbitrary"),
        ),
    )(xs, wT, biasT, basisT)
    return (jnp.swapaxes(outT, 1, 2), jnp.swapaxes(mapT, 1, 2))
